# baseline (speedup 1.0000x reference)
import jax
import jax.numpy as jnp
from jax.experimental import pallas as pl
from jax.experimental.pallas import tpu as pltpu

NUM_HEADS = 32
NUM_KV_HEADS = 8
HEAD_DIM = 128
BLOCK = 128
GATE_DIM = 64
NB = 16
NREP = NUM_HEADS // NUM_KV_HEADS
CW = 4
CWT = CW * BLOCK
QS = NREP * BLOCK
NEG = -1e30


def _gate_kernel(qb_ref, kb_ref, wq_ref, wk_ref, bias_ref):
    row = jax.lax.broadcasted_iota(jnp.int32, (NB, NB), 0)
    col = jax.lax.broadcasted_iota(jnp.int32, (NB, NB), 1)
    eye = row == col
    wq = wq_ref[...].astype(jnp.bfloat16)
    wk = wk_ref[...].astype(jnp.bfloat16)
    for h in range(NUM_HEADS):
        kvh = h // NREP
        hi = h % NREP
        gq = jnp.dot(qb_ref[h].astype(jnp.bfloat16), wq,
                     preferred_element_type=jnp.float32)
        gk = jnp.dot(kb_ref[h].astype(jnp.bfloat16), wk,
                     preferred_element_type=jnp.float32)
        logits = jax.lax.dot_general(
            gq.astype(jnp.bfloat16), gk.astype(jnp.bfloat16),
            (((1,), (1,)), ((), ())),
            preferred_element_type=jnp.float32) * 0.125
        active = (jax.nn.sigmoid(logits) > 0.5) | eye
        bias_ref[:, kvh, hi, :] = jnp.where(active, 0.0, NEG)


def _flash_kernel(q_ref, k_ref, v_ref, bias_ref, o_ref):
    qn = pl.program_id(1)

    q = jnp.concatenate(
        [q_ref[:, hi * HEAD_DIM:(hi + 1) * HEAD_DIM] for hi in range(NREP)],
        axis=0)

    b44 = bias_ref[0, 0].astype(jnp.bfloat16)
    er_r = jax.lax.broadcasted_iota(jnp.int32, (QS, NREP), 0) // BLOCK
    er_c = jax.lax.broadcasted_iota(jnp.int32, (QS, NREP), 1)
    e_r = (er_r == er_c).astype(jnp.bfloat16)
    b512 = jnp.dot(e_r, b44, preferred_element_type=jnp.float32).astype(
        jnp.bfloat16)

    def chunk(ch, carry, causal):
        m_p, l_p, acc = carry
        kc = k_ref[pl.ds(ch * CWT, CWT), :]
        vc = v_ref[pl.ds(ch * CWT, CWT), :]
        sij = jax.lax.dot_general(
            q, kc, (((1,), (1,)), ((), ())),
            preferred_element_type=jnp.float32)
        mrow = jax.lax.broadcasted_iota(jnp.int32, (NB, CWT), 0)
        colb = jax.lax.broadcasted_iota(jnp.int32, (NB, CWT), 1) // BLOCK
        e_c = (mrow == colb + CW * ch).astype(jnp.bfloat16)
        bias = jnp.dot(b512, e_c, preferred_element_type=jnp.float32)
        sij = sij + bias
        if causal:
            r = jax.lax.broadcasted_iota(jnp.int32, (QS, CWT), 0)
            c = jax.lax.broadcasted_iota(jnp.int32, (QS, CWT), 1)
            qtok = qn * BLOCK + jnp.bitwise_and(r, BLOCK - 1)
            ktok = ch * CWT + c
            sij = jnp.where(qtok >= ktok, sij, NEG)
        m_c = jnp.maximum(m_p, jnp.max(sij, axis=1, keepdims=True))
        alpha = jnp.exp(m_p - m_c)
        p = jnp.exp(sij - m_c)
        l_c = l_p * alpha + jnp.sum(p, axis=1, keepdims=True)
        acc = acc * alpha + jax.lax.dot(
            p.astype(v_ref.dtype), vc, preferred_element_type=jnp.float32)
        return m_c, l_c, acc

    init = (jnp.full((QS, 1), NEG, jnp.float32),
            jnp.zeros((QS, 1), jnp.float32),
            jnp.zeros((QS, HEAD_DIM), jnp.float32))
    nfull = qn // CW
    carry = jax.lax.fori_loop(
        0, nfull, lambda ch, c: chunk(ch, c, causal=False), init)
    m_f, l_f, acc = chunk(nfull, carry, causal=True)

    out = acc / l_f
    for hi in range(NREP):
        o_ref[:, hi * HEAD_DIM:(hi + 1) * HEAD_DIM] = \
            out[hi * BLOCK:(hi + 1) * BLOCK, :]


def kernel(query, key, value, Wq_g, Wk_g):
    T = query.shape[0]
    H, KVH, D, B = NUM_HEADS, NUM_KV_HEADS, HEAD_DIM, BLOCK

    q4 = query.reshape(1, T, H, D).transpose(0, 2, 1, 3)
    k4 = key.reshape(1, T, KVH, D).transpose(0, 2, 1, 3)
    k4 = jnp.repeat(k4, NREP, axis=1)
    qb = q4.reshape(1, H, NB, B, D).mean(axis=3)[0]
    kb = k4.reshape(1, H, NB, B, D).mean(axis=3)[0]

    bias = pl.pallas_call(
        _gate_kernel,
        in_specs=[
            pl.BlockSpec((H, NB, D), lambda: (0, 0, 0)),
            pl.BlockSpec((H, NB, D), lambda: (0, 0, 0)),
            pl.BlockSpec((D, GATE_DIM), lambda: (0, 0)),
            pl.BlockSpec((D, GATE_DIM), lambda: (0, 0)),
        ],
        out_specs=pl.BlockSpec((NB, KVH, NREP, NB),
                               lambda: (0, 0, 0, 0)),
        out_shape=jax.ShapeDtypeStruct((NB, KVH, NREP, NB), jnp.float32),
    )(qb, kb, Wq_g, Wk_g)

    scale = D ** -0.5
    qh = (query * scale).astype(jnp.bfloat16)
    kh = key.astype(jnp.bfloat16)
    vh = value.astype(jnp.bfloat16)

    out = pl.pallas_call(
        _flash_kernel,
        grid=(KVH, NB),
        in_specs=[
            pl.BlockSpec((B, NREP * D), lambda kvh, qn: (qn, kvh)),
            pl.BlockSpec((T, D), lambda kvh, qn: (0, kvh)),
            pl.BlockSpec((T, D), lambda kvh, qn: (0, kvh)),
            pl.BlockSpec((1, 1, NREP, NB), lambda kvh, qn: (qn, kvh, 0, 0)),
        ],
        out_specs=pl.BlockSpec((B, NREP * D), lambda kvh, qn: (qn, kvh)),
        out_shape=jax.ShapeDtypeStruct((T, H * D), jnp.float32),
        compiler_params=pltpu.CompilerParams(
            dimension_semantics=("arbitrary", "arbitrary")),
    )(qh, kh, vh, bias)
    return out
